# Initial kernel scaffold; baseline (speedup 1.0000x reference)
import jax
import jax.numpy as jnp
from jax.experimental import pallas as pl


def kernel(qk_dots, rp_buckets, rel_bias_table):
    raise NotImplementedError("write your pallas kernel here")


# speedup vs baseline: 24.7915x; 24.7915x over previous
import jax
import jax.numpy as jnp
from jax.experimental import pallas as pl
from jax.experimental.pallas import tpu as pltpu

_NUM_BUCKETS = 32


def _body(tab_ref, rb_ref, qk_ref, out_ref):
    rb = rb_ref[...]
    bias = jnp.full(rb.shape, tab_ref[0, 0, 0], dtype=jnp.float32)
    for k in range(1, _NUM_BUCKETS):
        bias = jnp.where(rb == k, tab_ref[0, 0, k], bias)
    out_ref[...] = qk_ref[...] + bias[None, None]


def kernel(qk_dots, rp_buckets, rel_bias_table):
    B, H, I, J = qk_dots.shape
    tab = (rel_bias_table + 1.0).T.reshape(H, 1, _NUM_BUCKETS)

    BI = 256
    grid = (H, I // BI)

    out = pl.pallas_call(
        _body,
        grid=grid,
        in_specs=[
            pl.BlockSpec((1, 1, _NUM_BUCKETS), lambda h, i: (h, 0, 0)),
            pl.BlockSpec((BI, J), lambda h, i: (i, 0)),
            pl.BlockSpec((1, 1, BI, J), lambda h, i: (0, h, i, 0)),
        ],
        out_specs=pl.BlockSpec((1, 1, BI, J), lambda h, i: (0, h, i, 0)),
        out_shape=jax.ShapeDtypeStruct(qk_dots.shape, qk_dots.dtype),
    )(tab, rp_buckets, qk_dots)
    return out
